# Initial kernel scaffold; baseline (speedup 1.0000x reference)
import jax
import jax.numpy as jnp
from jax.experimental import pallas as pl


def kernel(coords, edge_index, params):
    raise NotImplementedError("write your pallas kernel here")


# speedup vs baseline: 4.4400x; 4.4400x over previous
import functools

import jax
import jax.numpy as jnp
from jax import lax
from jax.experimental import pallas as pl
from jax.experimental.pallas import tpu as pltpu
from jax.experimental.pallas import tpu_sc as plsc

N = 10000
E = 320000
DIM = 128
HID = 64

NC = 2
NS = 16
NW = NC * NS
CHUNK = 80
WROWS = E // (CHUNK * NW)
NPAD = 10240
NODES_PER_TILE = NPAD // NS

_mesh = plsc.VectorSubcoreMesh(core_axis_name="c", subcore_axis_name="s")


def _make_segsum(D):
    @functools.partial(
        pl.kernel,
        out_type=jax.ShapeDtypeStruct((NC, NPAD, D), jnp.float32),
        mesh=_mesh,
        compiler_params=pltpu.CompilerParams(use_tc_tiling_on_sc=False),
        scratch_types=[
            pltpu.VMEM((WROWS, CHUNK), jnp.int32),
            pltpu.VMEM((WROWS, CHUNK), jnp.int32),
            pltpu.VMEM((CHUNK, D), jnp.float32),
            pltpu.VMEM_SHARED((NPAD, D), jnp.float32),
            pltpu.SemaphoreType.DMA,
        ],
    )
    def segsum(table_hbm, src_hbm, dst_hbm, zeros_hbm, out_hbm,
               idx_src, idx_dst, rows, acc, sem):
        c = lax.axis_index("c")
        s = lax.axis_index("s")
        w = c * NS + s

        pltpu.sync_copy(zeros_hbm, acc.at[pl.ds(s * NODES_PER_TILE, NODES_PER_TILE)])
        pltpu.sync_copy(src_hbm.at[w], idx_src)
        pltpu.sync_copy(dst_hbm.at[w], idx_dst)
        plsc.subcore_barrier()

        def body(i, _):
            pltpu.async_copy(table_hbm.at[idx_src.at[i]], rows, sem).wait()
            pltpu.sync_copy(rows, acc.at[idx_dst.at[i]], add=True)
            return 0

        lax.fori_loop(0, WROWS, body, 0)
        plsc.subcore_barrier()
        pltpu.sync_copy(
            acc.at[pl.ds(s * NODES_PER_TILE, NODES_PER_TILE)],
            out_hbm.at[c, pl.ds(s * NODES_PER_TILE, NODES_PER_TILE)],
        )

    return segsum


_segsum128 = _make_segsum(DIM)
_segsum64 = _make_segsum(HID)


@functools.partial(
    pl.kernel,
    out_type=(
        jax.ShapeDtypeStruct((E, HID), jnp.float32),
        jax.ShapeDtypeStruct((NW, 2, HID), jnp.float32),
    ),
    mesh=_mesh,
    compiler_params=pltpu.CompilerParams(use_tc_tiling_on_sc=False),
    scratch_types=[
        pltpu.VMEM((WROWS, CHUNK), jnp.int32),
        pltpu.VMEM((WROWS, CHUNK), jnp.int32),
        pltpu.VMEM((CHUNK, HID), jnp.float32),
        pltpu.VMEM((CHUNK, HID), jnp.float32),
        pltpu.VMEM((CHUNK, HID), jnp.float32),
        pltpu.VMEM((2, HID), jnp.float32),
        pltpu.SemaphoreType.DMA,
        pltpu.SemaphoreType.DMA,
    ],
)
def _edge_gather(p_hbm, q_hbm, src_hbm, dst_hbm, h_hbm, stats_hbm,
                 idx_src, idx_dst, rows_p, rows_q, hbuf, stats_buf, sem_p, sem_q):
    c = lax.axis_index("c")
    s = lax.axis_index("s")
    w = c * NS + s
    G = HID // 16

    pltpu.sync_copy(src_hbm.at[w], idx_src)
    pltpu.sync_copy(dst_hbm.at[w], idx_dst)

    def chunk_body(i, carry):
        dp = pltpu.async_copy(p_hbm.at[idx_src.at[i]], rows_p, sem_p)
        dq = pltpu.async_copy(q_hbm.at[idx_dst.at[i]], rows_q, sem_q)
        dp.wait()
        dq.wait()

        def row_body(r, car):
            sums, sqs = car
            new_sums, new_sqs = [], []
            for j in range(G):
                a = rows_p[r, pl.ds(j * 16, 16)]
                b = rows_q[r, pl.ds(j * 16, 16)]
                v = a + b
                hbuf[r, pl.ds(j * 16, 16)] = v
                new_sums.append(sums[j] + v)
                new_sqs.append(sqs[j] + v * v)
            return (tuple(new_sums), tuple(new_sqs))

        carry = lax.fori_loop(0, CHUNK, row_body, carry)
        pltpu.sync_copy(hbuf, h_hbm.at[pl.ds((w * WROWS + i) * CHUNK, CHUNK)])
        return carry

    zero = jnp.zeros((16,), jnp.float32)
    carry0 = (tuple(zero for _ in range(G)), tuple(zero for _ in range(G)))
    sums, sqs = lax.fori_loop(0, WROWS, chunk_body, carry0)
    for j in range(G):
        stats_buf[0, pl.ds(j * 16, 16)] = sums[j]
        stats_buf[1, pl.ds(j * 16, 16)] = sqs[j]
    pltpu.sync_copy(stats_buf, stats_hbm.at[w])


def _bn_act(h, g, b, eps=1e-5):
    m = jnp.mean(h, axis=0, keepdims=True)
    v = jnp.mean((h - m) ** 2, axis=0, keepdims=True)
    return jax.nn.relu(g * (h - m) * lax.rsqrt(v + eps) + b)


def _gin_block(z, pr):
    h = jnp.dot(z, pr["W1"], preferred_element_type=jnp.float32) + pr["b1"]
    h = _bn_act(h, pr["g1"], pr["be1"])
    h = jnp.dot(h, pr["W2"], preferred_element_type=jnp.float32) + pr["b2"]
    h = _bn_act(h, pr["g2"], pr["be2"])
    return jnp.dot(h, pr["W3"], preferred_element_type=jnp.float32) + pr["b3"]


def _tc_edge_head(h_ref, stats_ref, out_ref, *, mg1, mbe1, mW2, mb2, mW3, mb3):
    tot = jnp.sum(stats_ref[...], axis=0)
    m = tot[0:1, :] / E
    var = tot[1:2, :] / E - m * m
    inv = lax.rsqrt(var + 1e-5)
    hn = jax.nn.relu((h_ref[...] - m) * (inv * mg1) + mbe1)
    h2 = jax.nn.relu(jnp.dot(hn, mW2, preferred_element_type=jnp.float32) + mb2)
    out_ref[...] = jnp.dot(h2, mW3, preferred_element_type=jnp.float32) + mb3


def kernel(coords, edge_index, params):
    src2d = edge_index[0].reshape(NW, WROWS, CHUNK)
    dst2d = edge_index[1].reshape(NW, WROWS, CHUNK)
    zeros128 = jnp.zeros((NODES_PER_TILE, DIM), jnp.float32)
    zeros64 = jnp.zeros((NODES_PER_TILE, HID), jnp.float32)
    p = params

    agg0p = _segsum128(coords, src2d, dst2d, zeros128)[:, :N]
    x1 = _node_mlp(coords, agg0p, p["gin0"], p["bn0_g"], p["bn0_b"], DIM)

    agg1p = _segsum64(x1, src2d, dst2d, zeros64)[:, :N]
    pq = _node_mlp1(x1, agg1p, p["gin1"], p["bn1_g"], p["bn1_b"],
                    p["mW1"][:HID], p["mW1"][HID:])
    P, Q = pq

    h, stats = _edge_gather(P, Q, src2d, dst2d)
    out = _edge_head(h, stats, p)
    return out


def _node_mlp(coords, aggp, pr, bng, bnb, din):
    def body(coords_ref, aggp_ref, w1, b1, g1, be1, w2, b2, g2, be2, w3, b3,
             bg, bb, out_ref):
        z = coords_ref[...] + aggp_ref[0] + aggp_ref[1]
        prd = {"W1": w1[...], "b1": b1[...], "g1": g1[...], "be1": be1[...],
               "W2": w2[...], "b2": b2[...], "g2": g2[...], "be2": be2[...],
               "W3": w3[...], "b3": b3[...]}
        x = _gin_block(z, prd)
        out_ref[...] = _bn_act(x, bg[...], bb[...])

    args = (coords, aggp,
            pr["W1"], pr["b1"].reshape(1, -1), pr["g1"].reshape(1, -1),
            pr["be1"].reshape(1, -1), pr["W2"], pr["b2"].reshape(1, -1),
            pr["g2"].reshape(1, -1), pr["be2"].reshape(1, -1), pr["W3"],
            pr["b3"].reshape(1, -1), bng.reshape(1, -1), bnb.reshape(1, -1))
    return pl.pallas_call(
        body,
        out_shape=jax.ShapeDtypeStruct((N, HID), jnp.float32),
    )(*args)


def _node_mlp1(x1, aggp, pr, bng, bnb, w1a, w1b):
    def body(x_ref, aggp_ref, w1, b1, g1, be1, w2, b2, g2, be2, w3, b3,
             bg, bb, wa, wb, p_ref, q_ref):
        z = x_ref[...] + aggp_ref[0] + aggp_ref[1]
        prd = {"W1": w1[...], "b1": b1[...], "g1": g1[...], "be1": be1[...],
               "W2": w2[...], "b2": b2[...], "g2": g2[...], "be2": be2[...],
               "W3": w3[...], "b3": b3[...]}
        x = _gin_block(z, prd)
        x = _bn_act(x, bg[...], bb[...])
        p_ref[...] = jnp.dot(x, wa[...], preferred_element_type=jnp.float32)
        q_ref[...] = jnp.dot(x, wb[...], preferred_element_type=jnp.float32)

    args = (x1, aggp,
            pr["W1"], pr["b1"].reshape(1, -1), pr["g1"].reshape(1, -1),
            pr["be1"].reshape(1, -1), pr["W2"], pr["b2"].reshape(1, -1),
            pr["g2"].reshape(1, -1), pr["be2"].reshape(1, -1), pr["W3"],
            pr["b3"].reshape(1, -1), bng.reshape(1, -1), bnb.reshape(1, -1),
            w1a, w1b)
    return pl.pallas_call(
        body,
        out_shape=(jax.ShapeDtypeStruct((N, HID), jnp.float32),
                   jax.ShapeDtypeStruct((N, HID), jnp.float32)),
    )(*args)


def _edge_head(h, stats, p):
    EB = 5000
    nblk = E // EB

    def body(h_ref, stats_ref, mg1, mbe1, w2, b2, w3, b3, out_ref):
        _tc_edge_head(h_ref, stats_ref, out_ref,
                      mg1=mg1[...], mbe1=mbe1[...], mW2=w2[...], mb2=b2[...],
                      mW3=w3[...], mb3=b3[...])

    out2d = pl.pallas_call(
        body,
        grid=(nblk,),
        in_specs=[
            pl.BlockSpec((EB, HID), lambda i: (i, 0)),
            pl.BlockSpec((NW, 2, HID), lambda i: (0, 0, 0)),
            pl.BlockSpec((1, HID), lambda i: (0, 0)),
            pl.BlockSpec((1, HID), lambda i: (0, 0)),
            pl.BlockSpec((HID, HID // 2), lambda i: (0, 0)),
            pl.BlockSpec((1, HID // 2), lambda i: (0, 0)),
            pl.BlockSpec((HID // 2, 1), lambda i: (0, 0)),
            pl.BlockSpec((1, 1), lambda i: (0, 0)),
        ],
        out_specs=pl.BlockSpec((EB, 1), lambda i: (i, 0)),
        out_shape=jax.ShapeDtypeStruct((E, 1), jnp.float32),
    )(h, stats, p["mg1"].reshape(1, -1), p["mbe1"].reshape(1, -1),
      p["mW2"], p["mb2"].reshape(1, -1), p["mW3"], p["mb3"].reshape(1, 1))
    return out2d.reshape(E)


# speedup vs baseline: 7.2133x; 1.6246x over previous
import functools

from functools import partial

import jax
import jax.numpy as jnp
from jax import lax
from jax.experimental import pallas as pl
from jax.experimental.pallas import tpu as pltpu
from jax.experimental.pallas import tpu_sc as plsc

N = 10000
E = 320000
DIM = 128
HID = 64

NC = 2
NS = 16
NW = NC * NS
CHUNK = 80
WROWS = E // (CHUNK * NW)
NPAD = 10240
NODES_PER_TILE = NPAD // NS

def _dot(a, b):
    return jnp.dot(a, b, preferred_element_type=jnp.float32)


_mesh = plsc.VectorSubcoreMesh(core_axis_name="c", subcore_axis_name="s")
_sc_params = pltpu.CompilerParams(use_tc_tiling_on_sc=False)


def _make_segsum(D, NBUF):
    @functools.partial(
        pl.kernel,
        out_type=jax.ShapeDtypeStruct((NC, NPAD, D), jnp.float32),
        mesh=_mesh,
        compiler_params=_sc_params,
        scratch_types=[
            pltpu.VMEM((WROWS, CHUNK), jnp.int32),
            pltpu.VMEM((WROWS, CHUNK), jnp.int32),
        ] + [pltpu.VMEM((CHUNK, D), jnp.float32) for _ in range(NBUF)]
          + [pltpu.VMEM_SHARED((NPAD, D), jnp.float32)]
          + [pltpu.SemaphoreType.DMA for _ in range(2 * NBUF)],
    )
    def segsum(table_hbm, src_hbm, dst_hbm, zeros_hbm, out_hbm,
               idx_src, idx_dst, *bufs_acc_sems):
        rows = bufs_acc_sems[:NBUF]
        acc = bufs_acc_sems[NBUF]
        gs = bufs_acc_sems[NBUF + 1:NBUF + 1 + NBUF]
        ss = bufs_acc_sems[NBUF + 1 + NBUF:]
        c = lax.axis_index("c")
        s = lax.axis_index("s")
        w = c * NS + s

        def issue_g(j, b):
            return pltpu.async_copy(table_hbm.at[idx_src.at[j]], rows[b], gs[b])

        def issue_s(j, b):
            return pltpu.async_copy(rows[b], acc.at[idx_dst.at[j]], ss[b],
                                    add=True)

        pltpu.sync_copy(zeros_hbm, acc.at[pl.ds(s * NODES_PER_TILE, NODES_PER_TILE)])
        pltpu.sync_copy(src_hbm.at[w], idx_src)
        pltpu.sync_copy(dst_hbm.at[w], idx_dst)
        plsc.subcore_barrier()

        def round_body(r, _):
            j0 = r * NBUF
            gds = [issue_g(j0 + b, b) for b in range(NBUF)]
            sds = []
            for b in range(NBUF):
                gds[b].wait()
                sds.append(issue_s(j0 + b, b))
            for sd in sds:
                sd.wait()
            return 0

        nrounds = WROWS // NBUF
        lax.fori_loop(0, nrounds, round_body, 0)

        tds = []
        for jj in range(NBUF * nrounds, WROWS):
            b = jj - NBUF * nrounds
            tds.append((issue_g(jj, b), jj, b))
        sds = []
        for gd, jj, b in tds:
            gd.wait()
            sds.append(issue_s(jj, b))
        for sd in sds:
            sd.wait()

        plsc.subcore_barrier()
        pltpu.sync_copy(
            acc.at[pl.ds(s * NODES_PER_TILE, NODES_PER_TILE)],
            out_hbm.at[c, pl.ds(s * NODES_PER_TILE, NODES_PER_TILE)],
        )

    return segsum


_segsum128 = _make_segsum(DIM, 2)
_segsum64 = _make_segsum(HID, 6)

HROWS = CHUNK // 2


@functools.partial(
    pl.kernel,
    out_type=(
        jax.ShapeDtypeStruct((E // 2, 2 * HID), jnp.float32),
        jax.ShapeDtypeStruct((NW, 2, HID), jnp.float32),
    ),
    mesh=_mesh,
    compiler_params=_sc_params,
    scratch_types=[
        pltpu.VMEM((WROWS, CHUNK), jnp.int32),
        pltpu.VMEM((WROWS, CHUNK), jnp.int32),
        pltpu.VMEM((CHUNK, HID), jnp.float32),
        pltpu.VMEM((CHUNK, HID), jnp.float32),
        pltpu.VMEM((CHUNK, HID), jnp.float32),
        pltpu.VMEM((CHUNK, HID), jnp.float32),
        pltpu.VMEM((HROWS, 2 * HID), jnp.float32),
        pltpu.VMEM((HROWS, 2 * HID), jnp.float32),
        pltpu.VMEM((2, HID), jnp.float32),
        pltpu.SemaphoreType.DMA,
        pltpu.SemaphoreType.DMA,
        pltpu.SemaphoreType.DMA,
        pltpu.SemaphoreType.DMA,
        pltpu.SemaphoreType.DMA,
        pltpu.SemaphoreType.DMA,
    ],
)
def _edge_gather(p_hbm, q_hbm, src_hbm, dst_hbm, h_hbm, stats_hbm,
                 idx_src, idx_dst, rp0, rp1, rq0, rq1, hb0, hb1, stats_buf,
                 sp0, sp1, sq0, sq1, sw0, sw1):
    rows_p = (rp0, rp1)
    rows_q = (rq0, rq1)
    hbuf = (hb0, hb1)
    sems_p = (sp0, sp1)
    sems_q = (sq0, sq1)
    sems_w = (sw0, sw1)
    c = lax.axis_index("c")
    s = lax.axis_index("s")
    w = c * NS + s
    G = HID // 16

    pltpu.sync_copy(src_hbm.at[w], idx_src)
    pltpu.sync_copy(dst_hbm.at[w], idx_dst)

    def issue_gathers(j, b):
        dp = pltpu.async_copy(p_hbm.at[idx_src.at[j]], rows_p[b], sems_p[b])
        dq = pltpu.async_copy(q_hbm.at[idx_dst.at[j]], rows_q[b], sems_q[b])
        return dp, dq

    def issue_w(j, b):
        return pltpu.async_copy(
            hbuf[b], h_hbm.at[pl.ds((w * WROWS + j) * HROWS, HROWS)], sems_w[b])

    def compute(b, carry):
        def row_body(rr, car):
            sums, sqs = car
            new_sums = list(sums)
            new_sqs = list(sqs)
            for half in range(2):
                r = 2 * rr + half
                for g in range(G):
                    a = rows_p[b][r, pl.ds(g * 16, 16)]
                    bb = rows_q[b][r, pl.ds(g * 16, 16)]
                    v = a + bb
                    hbuf[b][rr, pl.ds(half * HID + g * 16, 16)] = v
                    new_sums[g] = new_sums[g] + v
                    new_sqs[g] = new_sqs[g] + v * v
            return (tuple(new_sums), tuple(new_sqs))

        return lax.fori_loop(0, HROWS, row_body, carry)

    zero = jnp.zeros((16,), jnp.float32)
    carry = (tuple(zero for _ in range(G)), tuple(zero for _ in range(G)))

    def pair_body(r, car):
        j0 = 2 * r
        dp0, dq0 = issue_gathers(j0, 0)
        dp1, dq1 = issue_gathers(j0 + 1, 1)
        dp0.wait()
        dq0.wait()
        car = compute(0, car)
        w0 = issue_w(j0, 0)
        dp1.wait()
        dq1.wait()
        car = compute(1, car)
        w1 = issue_w(j0 + 1, 1)
        w0.wait()
        w1.wait()
        return car

    carry = lax.fori_loop(0, WROWS // 2, pair_body, carry)

    jj = WROWS - 1
    dp, dq = issue_gathers(jj, 0)
    dp.wait()
    dq.wait()
    carry = compute(0, carry)
    issue_w(jj, 0).wait()

    sums, sqs = carry
    for g in range(G):
        stats_buf[0, pl.ds(g * 16, 16)] = sums[g]
        stats_buf[1, pl.ds(g * 16, 16)] = sqs[g]
    pltpu.sync_copy(stats_buf, stats_hbm.at[w])


def _bn_act(h, g, b, eps=1e-5):
    m = jnp.mean(h, axis=0, keepdims=True)
    v = jnp.mean((h - m) ** 2, axis=0, keepdims=True)
    return jax.nn.relu(g * (h - m) * lax.rsqrt(v + eps) + b)


def _gin_block(z, pr):
    h = _dot(z, pr["W1"]) + pr["b1"]
    h = _bn_act(h, pr["g1"], pr["be1"])
    h = _dot(h, pr["W2"]) + pr["b2"]
    h = _bn_act(h, pr["g2"], pr["be2"])
    return _dot(h, pr["W3"]) + pr["b3"]


def _node_mlp(coords, aggp, pr, bng, bnb):
    def body(coords_ref, aggp_ref, w1, b1, g1, be1, w2, b2, g2, be2, w3, b3,
             bg, bb, out_ref):
        z = coords_ref[...] + aggp_ref[0, :N] + aggp_ref[1, :N]
        prd = {"W1": w1[...], "b1": b1[...], "g1": g1[...], "be1": be1[...],
               "W2": w2[...], "b2": b2[...], "g2": g2[...], "be2": be2[...],
               "W3": w3[...], "b3": b3[...]}
        x = _gin_block(z, prd)
        out_ref[...] = _bn_act(x, bg[...], bb[...])

    args = (coords, aggp,
            pr["W1"], pr["b1"].reshape(1, -1), pr["g1"].reshape(1, -1),
            pr["be1"].reshape(1, -1), pr["W2"], pr["b2"].reshape(1, -1),
            pr["g2"].reshape(1, -1), pr["be2"].reshape(1, -1), pr["W3"],
            pr["b3"].reshape(1, -1), bng.reshape(1, -1), bnb.reshape(1, -1))
    return pl.pallas_call(
        body,
        out_shape=jax.ShapeDtypeStruct((N, HID), jnp.float32),
    )(*args)


def _node_mlp1(x1, aggp, pr, bng, bnb, w1a, w1b):
    def body(x_ref, aggp_ref, w1, b1, g1, be1, w2, b2, g2, be2, w3, b3,
             bg, bb, wa, wb, p_ref, q_ref):
        z = x_ref[...] + aggp_ref[0, :N] + aggp_ref[1, :N]
        prd = {"W1": w1[...], "b1": b1[...], "g1": g1[...], "be1": be1[...],
               "W2": w2[...], "b2": b2[...], "g2": g2[...], "be2": be2[...],
               "W3": w3[...], "b3": b3[...]}
        x = _gin_block(z, prd)
        x = _bn_act(x, bg[...], bb[...])
        p_ref[...] = _dot(x, wa[...])
        q_ref[...] = _dot(x, wb[...])

    args = (x1, aggp,
            pr["W1"], pr["b1"].reshape(1, -1), pr["g1"].reshape(1, -1),
            pr["be1"].reshape(1, -1), pr["W2"], pr["b2"].reshape(1, -1),
            pr["g2"].reshape(1, -1), pr["be2"].reshape(1, -1), pr["W3"],
            pr["b3"].reshape(1, -1), bng.reshape(1, -1), bnb.reshape(1, -1),
            w1a, w1b)
    return pl.pallas_call(
        body,
        out_shape=(jax.ShapeDtypeStruct((N, HID), jnp.float32),
                   jax.ShapeDtypeStruct((N, HID), jnp.float32)),
    )(*args)


def _edge_head(h2d, stats, p):
    EB = 4000
    nblk = (E // 2) // EB

    mW2, mb2, mW3, mb3 = p["mW2"], p["mb2"], p["mW3"], p["mb3"]
    w2bd = jnp.zeros((2 * HID, HID), jnp.float32)
    w2bd = w2bd.at[:HID, :HID // 2].set(mW2).at[HID:, HID // 2:].set(mW2)
    w3bd = jnp.zeros((HID, 2), jnp.float32)
    w3bd = w3bd.at[:HID // 2, 0].set(mW3[:, 0]).at[HID // 2:, 1].set(mW3[:, 0])
    b2t = jnp.concatenate([mb2, mb2]).reshape(1, HID)
    g1t = jnp.concatenate([p["mg1"], p["mg1"]]).reshape(1, 2 * HID)
    be1t = jnp.concatenate([p["mbe1"], p["mbe1"]]).reshape(1, 2 * HID)

    def body(h_ref, stats_ref, g1_ref, be1_ref, w2_ref, b2_ref, w3_ref, b3_ref,
             out_ref):
        tot = jnp.sum(stats_ref[...], axis=0)
        m = tot[0:1, :] / E
        var = tot[1:2, :] / E - m * m
        inv = lax.rsqrt(var + 1e-5)
        m2 = jnp.concatenate([m, m], axis=1)
        inv2 = jnp.concatenate([inv, inv], axis=1)
        hn = jax.nn.relu((h_ref[...] - m2) * (inv2 * g1_ref[...]) + be1_ref[...])
        h2 = jax.nn.relu(
            _dot(hn, w2_ref[...])
            + b2_ref[...])
        out_ref[...] = (_dot(h2, w3_ref[...])
                        + b3_ref[0, 0])

    out2d = pl.pallas_call(
        body,
        grid=(nblk,),
        in_specs=[
            pl.BlockSpec((EB, 2 * HID), lambda i: (i, 0)),
            pl.BlockSpec((NW, 2, HID), lambda i: (0, 0, 0)),
            pl.BlockSpec((1, 2 * HID), lambda i: (0, 0)),
            pl.BlockSpec((1, 2 * HID), lambda i: (0, 0)),
            pl.BlockSpec((2 * HID, HID), lambda i: (0, 0)),
            pl.BlockSpec((1, HID), lambda i: (0, 0)),
            pl.BlockSpec((HID, 2), lambda i: (0, 0)),
            pl.BlockSpec((1, 1), lambda i: (0, 0)),
        ],
        out_specs=pl.BlockSpec((EB, 2), lambda i: (i, 0)),
        out_shape=jax.ShapeDtypeStruct((E // 2, 2), jnp.float32),
    )(h2d, stats, g1t, be1t, w2bd, b2t, w3bd, mb3.reshape(1, 1))
    return out2d.reshape(E)


def kernel(coords, edge_index, params):
    src2d = edge_index[0].reshape(NW, WROWS, CHUNK)
    dst2d = edge_index[1].reshape(NW, WROWS, CHUNK)
    zeros128 = jnp.zeros((NODES_PER_TILE, DIM), jnp.float32)
    zeros64 = jnp.zeros((NODES_PER_TILE, HID), jnp.float32)
    p = params

    agg0p = _segsum128(coords, src2d, dst2d, zeros128)
    x1 = _node_mlp(coords, agg0p, p["gin0"], p["bn0_g"], p["bn0_b"])

    agg1p = _segsum64(x1, src2d, dst2d, zeros64)
    P, Q = _node_mlp1(x1, agg1p, p["gin1"], p["bn1_g"], p["bn1_b"],
                      p["mW1"][:HID], p["mW1"][HID:])

    h2d, stats = _edge_gather(P, Q, src2d, dst2d)
    return _edge_head(h2d, stats, p)


# speedup vs baseline: 7.3495x; 1.0189x over previous
import functools

from functools import partial

import jax
import jax.numpy as jnp
from jax import lax
from jax.experimental import pallas as pl
from jax.experimental.pallas import tpu as pltpu
from jax.experimental.pallas import tpu_sc as plsc

N = 10000
E = 320000
DIM = 128
HID = 64

NC = 2
NS = 16
NW = NC * NS
CHUNK = 80
WROWS = E // (CHUNK * NW)
NPAD = 10240
NODES_PER_TILE = NPAD // NS

def _dot(a, b):
    return jnp.dot(a, b, preferred_element_type=jnp.float32)


_mesh = plsc.VectorSubcoreMesh(core_axis_name="c", subcore_axis_name="s")
_sc_params = pltpu.CompilerParams(use_tc_tiling_on_sc=False)


def _make_segsum(D, NBUF):
    @functools.partial(
        pl.kernel,
        out_type=jax.ShapeDtypeStruct((NC, NPAD, D), jnp.float32),
        mesh=_mesh,
        compiler_params=_sc_params,
        scratch_types=[
            pltpu.VMEM((WROWS, CHUNK), jnp.int32),
            pltpu.VMEM((WROWS, CHUNK), jnp.int32),
        ] + [pltpu.VMEM((CHUNK, D), jnp.float32) for _ in range(NBUF)]
          + [pltpu.VMEM_SHARED((NPAD, D), jnp.float32)]
          + [pltpu.SemaphoreType.DMA for _ in range(2 * NBUF)],
    )
    def segsum(table_hbm, src_hbm, dst_hbm, zeros_hbm, out_hbm,
               idx_src, idx_dst, *bufs_acc_sems):
        rows = bufs_acc_sems[:NBUF]
        acc = bufs_acc_sems[NBUF]
        gs = bufs_acc_sems[NBUF + 1:NBUF + 1 + NBUF]
        ss = bufs_acc_sems[NBUF + 1 + NBUF:]
        c = lax.axis_index("c")
        s = lax.axis_index("s")
        w = c * NS + s

        def issue_g(j, b):
            return pltpu.async_copy(table_hbm.at[idx_src.at[j]], rows[b], gs[b])

        def issue_s(j, b):
            return pltpu.async_copy(rows[b], acc.at[idx_dst.at[j]], ss[b],
                                    add=True)

        pltpu.sync_copy(zeros_hbm, acc.at[pl.ds(s * NODES_PER_TILE, NODES_PER_TILE)])
        pltpu.sync_copy(src_hbm.at[w], idx_src)
        pltpu.sync_copy(dst_hbm.at[w], idx_dst)
        plsc.subcore_barrier()

        def round_body(r, _):
            j0 = r * NBUF
            gds = [issue_g(j0 + b, b) for b in range(NBUF)]
            sds = []
            for b in range(NBUF):
                gds[b].wait()
                sds.append(issue_s(j0 + b, b))
            for sd in sds:
                sd.wait()
            return 0

        nrounds = WROWS // NBUF
        lax.fori_loop(0, nrounds, round_body, 0)

        tds = []
        for jj in range(NBUF * nrounds, WROWS):
            b = jj - NBUF * nrounds
            tds.append((issue_g(jj, b), jj, b))
        sds = []
        for gd, jj, b in tds:
            gd.wait()
            sds.append(issue_s(jj, b))
        for sd in sds:
            sd.wait()

        plsc.subcore_barrier()
        pltpu.sync_copy(
            acc.at[pl.ds(s * NODES_PER_TILE, NODES_PER_TILE)],
            out_hbm.at[c, pl.ds(s * NODES_PER_TILE, NODES_PER_TILE)],
        )

    return segsum


_segsum64 = _make_segsum(HID, 6)

WROWS2 = E // (CHUNK * NS)
NBUF2 = 8


@functools.partial(
    pl.kernel,
    out_type=jax.ShapeDtypeStruct((NC, NPAD, HID), jnp.float32),
    mesh=_mesh,
    compiler_params=_sc_params,
    scratch_types=[
        pltpu.VMEM((WROWS2, CHUNK), jnp.int32),
        pltpu.VMEM((WROWS2, CHUNK), jnp.int32),
    ] + [pltpu.VMEM((CHUNK, HID), jnp.float32) for _ in range(NBUF2)]
      + [pltpu.VMEM_SHARED((NPAD, HID), jnp.float32)]
      + [pltpu.SemaphoreType.DMA for _ in range(2 * NBUF2)],
)
def _segsum_split(tab2_hbm, src_hbm, dst_hbm, zeros_hbm, out_hbm,
                  idx_src, idx_dst, *bufs_acc_sems):
    rows = bufs_acc_sems[:NBUF2]
    acc = bufs_acc_sems[NBUF2]
    gs = bufs_acc_sems[NBUF2 + 1:NBUF2 + 1 + NBUF2]
    ss = bufs_acc_sems[NBUF2 + 1 + NBUF2:]
    c = lax.axis_index("c")
    s = lax.axis_index("s")

    def issue_g(j, b):
        return pltpu.async_copy(tab2_hbm.at[idx_src.at[j]], rows[b], gs[b])

    def issue_s(j, b):
        return pltpu.async_copy(rows[b], acc.at[idx_dst.at[j]], ss[b], add=True)

    pltpu.sync_copy(zeros_hbm, acc.at[pl.ds(s * NODES_PER_TILE, NODES_PER_TILE)])
    pltpu.sync_copy(src_hbm.at[c, s], idx_src)
    pltpu.sync_copy(dst_hbm.at[s], idx_dst)
    plsc.subcore_barrier()

    def round_body(r, _):
        j0 = r * NBUF2
        gds = [issue_g(j0 + b, b) for b in range(NBUF2)]
        sds = []
        for b in range(NBUF2):
            gds[b].wait()
            sds.append(issue_s(j0 + b, b))
        for sd in sds:
            sd.wait()
        return 0

    nrounds = WROWS2 // NBUF2
    lax.fori_loop(0, nrounds, round_body, 0)

    tds = []
    for jj in range(NBUF2 * nrounds, WROWS2):
        b = jj - NBUF2 * nrounds
        tds.append((issue_g(jj, b), jj, b))
    sds = []
    for gd, jj, b in tds:
        gd.wait()
        sds.append(issue_s(jj, b))
    for sd in sds:
        sd.wait()

    plsc.subcore_barrier()
    pltpu.sync_copy(
        acc.at[pl.ds(s * NODES_PER_TILE, NODES_PER_TILE)],
        out_hbm.at[c, pl.ds(s * NODES_PER_TILE, NODES_PER_TILE)],
    )

HROWS = CHUNK // 2
EBUF = 4


@functools.partial(
    pl.kernel,
    out_type=(
        jax.ShapeDtypeStruct((E // 2, 2 * HID), jnp.float32),
        jax.ShapeDtypeStruct((NW, 2, HID), jnp.float32),
    ),
    mesh=_mesh,
    compiler_params=_sc_params,
    scratch_types=[
        pltpu.VMEM((WROWS, CHUNK), jnp.int32),
        pltpu.VMEM((WROWS, CHUNK), jnp.int32),
    ] + [pltpu.VMEM((CHUNK, HID), jnp.float32) for _ in range(2 * EBUF)]
      + [pltpu.VMEM((HROWS, 2 * HID), jnp.float32) for _ in range(EBUF)]
      + [pltpu.VMEM((2, HID), jnp.float32)]
      + [pltpu.SemaphoreType.DMA for _ in range(3 * EBUF)],
)
def _edge_gather(p_hbm, q_hbm, src_hbm, dst_hbm, h_hbm, stats_hbm,
                 idx_src, idx_dst, *bufs_sems):
    rows_p = bufs_sems[0:EBUF]
    rows_q = bufs_sems[EBUF:2 * EBUF]
    hbuf = bufs_sems[2 * EBUF:3 * EBUF]
    stats_buf = bufs_sems[3 * EBUF]
    sems_p = bufs_sems[3 * EBUF + 1:3 * EBUF + 1 + EBUF]
    sems_q = bufs_sems[3 * EBUF + 1 + EBUF:3 * EBUF + 1 + 2 * EBUF]
    sems_w = bufs_sems[3 * EBUF + 1 + 2 * EBUF:]
    c = lax.axis_index("c")
    s = lax.axis_index("s")
    w = c * NS + s
    G = HID // 16

    pltpu.sync_copy(src_hbm.at[w], idx_src)
    pltpu.sync_copy(dst_hbm.at[w], idx_dst)

    def issue_gathers(j, b):
        dp = pltpu.async_copy(p_hbm.at[idx_src.at[j]], rows_p[b], sems_p[b])
        dq = pltpu.async_copy(q_hbm.at[idx_dst.at[j]], rows_q[b], sems_q[b])
        return dp, dq

    def issue_w(j, b):
        return pltpu.async_copy(
            hbuf[b], h_hbm.at[pl.ds((w * WROWS + j) * HROWS, HROWS)], sems_w[b])

    def compute(b, carry):
        def row_body(rr, car):
            sums, sqs = car
            new_sums = list(sums)
            new_sqs = list(sqs)
            for half in range(2):
                r = 2 * rr + half
                for g in range(G):
                    a = rows_p[b][r, pl.ds(g * 16, 16)]
                    bb = rows_q[b][r, pl.ds(g * 16, 16)]
                    v = a + bb
                    hbuf[b][rr, pl.ds(half * HID + g * 16, 16)] = v
                    new_sums[g] = new_sums[g] + v
                    new_sqs[g] = new_sqs[g] + v * v
            return (tuple(new_sums), tuple(new_sqs))

        return lax.fori_loop(0, HROWS, row_body, carry)

    zero = jnp.zeros((16,), jnp.float32)
    carry = (tuple(zero for _ in range(G)), tuple(zero for _ in range(G)))

    def round_body(r, car):
        j0 = EBUF * r
        gds = [issue_gathers(j0 + b, b) for b in range(EBUF)]
        wds = []
        for b in range(EBUF):
            gds[b][0].wait()
            gds[b][1].wait()
            car = compute(b, car)
            wds.append(issue_w(j0 + b, b))
        for wd in wds:
            wd.wait()
        return car

    nrounds = WROWS // EBUF
    carry = lax.fori_loop(0, nrounds, round_body, carry)

    gds = []
    for jj in range(EBUF * nrounds, WROWS):
        gds.append((issue_gathers(jj, jj - EBUF * nrounds), jj))
    wds = []
    for (dp, dq), jj in gds:
        b = jj - EBUF * nrounds
        dp.wait()
        dq.wait()
        carry = compute(b, carry)
        wds.append(issue_w(jj, b))
    for wd in wds:
        wd.wait()

    sums, sqs = carry
    for g in range(G):
        stats_buf[0, pl.ds(g * 16, 16)] = sums[g]
        stats_buf[1, pl.ds(g * 16, 16)] = sqs[g]
    pltpu.sync_copy(stats_buf, stats_hbm.at[w])


def _bn_act(h, g, b, eps=1e-5):
    m = jnp.mean(h, axis=0, keepdims=True)
    v = jnp.mean((h - m) ** 2, axis=0, keepdims=True)
    return jax.nn.relu(g * (h - m) * lax.rsqrt(v + eps) + b)


def _gin_block(z, pr):
    h = _dot(z, pr["W1"]) + pr["b1"]
    h = _bn_act(h, pr["g1"], pr["be1"])
    h = _dot(h, pr["W2"]) + pr["b2"]
    h = _bn_act(h, pr["g2"], pr["be2"])
    return _dot(h, pr["W3"]) + pr["b3"]


def _node_mlp(coords, aggp, pr, bng, bnb):
    def body(coords_ref, aggp_ref, w1, b1, g1, be1, w2, b2, g2, be2, w3, b3,
             bg, bb, out_ref):
        agg = jnp.concatenate([aggp_ref[0, :N], aggp_ref[1, :N]], axis=1)
        z = coords_ref[...] + agg
        prd = {"W1": w1[...], "b1": b1[...], "g1": g1[...], "be1": be1[...],
               "W2": w2[...], "b2": b2[...], "g2": g2[...], "be2": be2[...],
               "W3": w3[...], "b3": b3[...]}
        x = _gin_block(z, prd)
        out_ref[...] = _bn_act(x, bg[...], bb[...])

    args = (coords, aggp,
            pr["W1"], pr["b1"].reshape(1, -1), pr["g1"].reshape(1, -1),
            pr["be1"].reshape(1, -1), pr["W2"], pr["b2"].reshape(1, -1),
            pr["g2"].reshape(1, -1), pr["be2"].reshape(1, -1), pr["W3"],
            pr["b3"].reshape(1, -1), bng.reshape(1, -1), bnb.reshape(1, -1))
    return pl.pallas_call(
        body,
        out_shape=jax.ShapeDtypeStruct((N, HID), jnp.float32),
    )(*args)


def _node_mlp1(x1, aggp, pr, bng, bnb, w1a, w1b):
    def body(x_ref, aggp_ref, w1, b1, g1, be1, w2, b2, g2, be2, w3, b3,
             bg, bb, wa, wb, p_ref, q_ref):
        z = x_ref[...] + aggp_ref[0, :N] + aggp_ref[1, :N]
        prd = {"W1": w1[...], "b1": b1[...], "g1": g1[...], "be1": be1[...],
               "W2": w2[...], "b2": b2[...], "g2": g2[...], "be2": be2[...],
               "W3": w3[...], "b3": b3[...]}
        x = _gin_block(z, prd)
        x = _bn_act(x, bg[...], bb[...])
        p_ref[...] = _dot(x, wa[...])
        q_ref[...] = _dot(x, wb[...])

    args = (x1, aggp,
            pr["W1"], pr["b1"].reshape(1, -1), pr["g1"].reshape(1, -1),
            pr["be1"].reshape(1, -1), pr["W2"], pr["b2"].reshape(1, -1),
            pr["g2"].reshape(1, -1), pr["be2"].reshape(1, -1), pr["W3"],
            pr["b3"].reshape(1, -1), bng.reshape(1, -1), bnb.reshape(1, -1),
            w1a, w1b)
    return pl.pallas_call(
        body,
        out_shape=(jax.ShapeDtypeStruct((N, HID), jnp.float32),
                   jax.ShapeDtypeStruct((N, HID), jnp.float32)),
    )(*args)


def _edge_head(h2d, stats, p):
    EB = 4000
    nblk = (E // 2) // EB

    mW2, mb2, mW3, mb3 = p["mW2"], p["mb2"], p["mW3"], p["mb3"]
    w2bd = jnp.zeros((2 * HID, HID), jnp.float32)
    w2bd = w2bd.at[:HID, :HID // 2].set(mW2).at[HID:, HID // 2:].set(mW2)
    w3bd = jnp.zeros((HID, 2), jnp.float32)
    w3bd = w3bd.at[:HID // 2, 0].set(mW3[:, 0]).at[HID // 2:, 1].set(mW3[:, 0])
    b2t = jnp.concatenate([mb2, mb2]).reshape(1, HID)
    g1t = jnp.concatenate([p["mg1"], p["mg1"]]).reshape(1, 2 * HID)
    be1t = jnp.concatenate([p["mbe1"], p["mbe1"]]).reshape(1, 2 * HID)

    def body(h_ref, stats_ref, g1_ref, be1_ref, w2_ref, b2_ref, w3_ref, b3_ref,
             out_ref):
        tot = jnp.sum(stats_ref[...], axis=0)
        m = tot[0:1, :] / E
        var = tot[1:2, :] / E - m * m
        inv = lax.rsqrt(var + 1e-5)
        m2 = jnp.concatenate([m, m], axis=1)
        inv2 = jnp.concatenate([inv, inv], axis=1)
        hn = jax.nn.relu((h_ref[...] - m2) * (inv2 * g1_ref[...]) + be1_ref[...])
        h2 = jax.nn.relu(
            _dot(hn, w2_ref[...])
            + b2_ref[...])
        out_ref[...] = (_dot(h2, w3_ref[...])
                        + b3_ref[0, 0])

    out2d = pl.pallas_call(
        body,
        grid=(nblk,),
        in_specs=[
            pl.BlockSpec((EB, 2 * HID), lambda i: (i, 0)),
            pl.BlockSpec((NW, 2, HID), lambda i: (0, 0, 0)),
            pl.BlockSpec((1, 2 * HID), lambda i: (0, 0)),
            pl.BlockSpec((1, 2 * HID), lambda i: (0, 0)),
            pl.BlockSpec((2 * HID, HID), lambda i: (0, 0)),
            pl.BlockSpec((1, HID), lambda i: (0, 0)),
            pl.BlockSpec((HID, 2), lambda i: (0, 0)),
            pl.BlockSpec((1, 1), lambda i: (0, 0)),
        ],
        out_specs=pl.BlockSpec((EB, 2), lambda i: (i, 0)),
        out_shape=jax.ShapeDtypeStruct((E // 2, 2), jnp.float32),
    )(h2d, stats, g1t, be1t, w2bd, b2t, w3bd, mb3.reshape(1, 1))
    return out2d.reshape(E)


def kernel(coords, edge_index, params):
    src2d = edge_index[0].reshape(NW, WROWS, CHUNK)
    dst2d = edge_index[1].reshape(NW, WROWS, CHUNK)
    src_ns = edge_index[0].reshape(NS, WROWS2, CHUNK)
    srcs2 = jnp.stack([src_ns, src_ns + N])
    dst_ns = edge_index[1].reshape(NS, WROWS2, CHUNK)
    tab2 = jnp.concatenate([coords[:, :HID], coords[:, HID:]], axis=0)
    zeros64 = jnp.zeros((NODES_PER_TILE, HID), jnp.float32)
    p = params

    agg0p = _segsum_split(tab2, srcs2, dst_ns, zeros64)
    x1 = _node_mlp(coords, agg0p, p["gin0"], p["bn0_g"], p["bn0_b"])

    agg1p = _segsum64(x1, src2d, dst2d, zeros64)
    P, Q = _node_mlp1(x1, agg1p, p["gin1"], p["bn1_g"], p["bn1_b"],
                      p["mW1"][:HID], p["mW1"][HID:])

    h2d, stats = _edge_gather(P, Q, src2d, dst2d)
    return _edge_head(h2d, stats, p)
